# baseline (speedup 1.0000x reference)
import numpy as np
import jax
import jax.numpy as jnp
from jax import lax
from jax.experimental import pallas as pl
from jax.experimental.pallas import tpu as pltpu
from jax.experimental.pallas import tpu_sc as plsc

_LOG_K = float(np.log(64))
_NW = 32
_CH = 256


def _make_sc_body(off):
  def _sc_body(u_hbm, v_hbm, b_hbm, ts_hbm, la_hbm, l1_hbm,
               uv_hbm, ut_hbm, c1_hbm, lai_hbm, l1i_hbm,
               u_vm0, u_vm1, v_vm0, v_vm1, b_vm0, b_vm1,
               ts_vm, la_vm, l1_vm,
               uv_vm0, uv_vm1, ut_vm0, ut_vm1, c1_vm0, c1_vm1,
               lai_vm0, lai_vm1, l1i_vm0, l1i_vm1,
               in_sem0, in_sem1, out_sem0, out_sem1):
    n = uv_hbm.shape[0]
    per_w = n // _NW
    nchunks = per_w // _CH
    wid = lax.axis_index("s") * 2 + lax.axis_index("c")
    base_w = wid * per_w

    u_vm = (u_vm0, u_vm1)
    v_vm = (v_vm0, v_vm1)
    b_vm = (b_vm0, b_vm1)
    uv_vm = (uv_vm0, uv_vm1)
    ut_vm = (ut_vm0, ut_vm1)
    c1_vm = (c1_vm0, c1_vm1)
    lai_vm = (lai_vm0, lai_vm1)
    l1i_vm = (l1i_vm0, l1i_vm1)
    in_sem = (in_sem0, in_sem1)
    out_sem = (out_sem0, out_sem1)

    pltpu.sync_copy(ts_hbm, ts_vm)
    pltpu.sync_copy(la_hbm, la_vm)
    pltpu.sync_copy(l1_hbm, l1_vm)

    def fire_in(b, base):
        pltpu.async_copy(u_hbm.at[pl.ds(base + off, _CH)], u_vm[b], in_sem[b])
        pltpu.async_copy(v_hbm.at[pl.ds(base + off, _CH)], v_vm[b], in_sem[b])
        pltpu.async_copy(b_hbm.at[pl.ds(base + off, _CH)], b_vm[b], in_sem[b])

    def wait_in(b, base):
        pltpu.make_async_copy(u_hbm.at[pl.ds(base + off, _CH)], u_vm[b],
                              in_sem[b]).wait()
        pltpu.make_async_copy(v_hbm.at[pl.ds(base + off, _CH)], v_vm[b],
                              in_sem[b]).wait()
        pltpu.make_async_copy(b_hbm.at[pl.ds(base + off, _CH)], b_vm[b],
                              in_sem[b]).wait()

    def fire_out(b, base):
        pltpu.async_copy(uv_vm[b], uv_hbm.at[pl.ds(base, _CH)], out_sem[b])
        pltpu.async_copy(ut_vm[b], ut_hbm.at[pl.ds(base, _CH)], out_sem[b])
        pltpu.async_copy(c1_vm[b], c1_hbm.at[pl.ds(base, _CH)], out_sem[b])
        pltpu.async_copy(lai_vm[b], lai_hbm.at[pl.ds(base, _CH)], out_sem[b])
        pltpu.async_copy(l1i_vm[b], l1i_hbm.at[pl.ds(base, _CH)], out_sem[b])

    def wait_out(b, base):
        pltpu.make_async_copy(uv_vm[b], uv_hbm.at[pl.ds(base, _CH)],
                              out_sem[b]).wait()
        pltpu.make_async_copy(ut_vm[b], ut_hbm.at[pl.ds(base, _CH)],
                              out_sem[b]).wait()
        pltpu.make_async_copy(c1_vm[b], c1_hbm.at[pl.ds(base, _CH)],
                              out_sem[b]).wait()
        pltpu.make_async_copy(lai_vm[b], lai_hbm.at[pl.ds(base, _CH)],
                              out_sem[b]).wait()
        pltpu.make_async_copy(l1i_vm[b], l1i_hbm.at[pl.ds(base, _CH)],
                              out_sem[b]).wait()

    def compute_chunk(b):
        def group(g, carry):
            rows = lax.iota(jnp.int32, 16) + (g * 16)
            v_vec = v_vm[b][pl.ds(g * 16, 16)]
            b_vec = b_vm[b][pl.ds(g * 16, 16)]
            t_vec = plsc.load_gather(ts_vm, [b_vec])
            la_vec = plsc.load_gather(la_vm, [t_vec])
            l1_vec = plsc.load_gather(l1_vm, [t_vec])
            uv_vec = plsc.load_gather(u_vm[b], [rows, v_vec])
            plsc.store_scatter(u_vm[b], [rows, v_vec],
                               jnp.full((16,), -1.0, jnp.float32))
            ms, cs = [], []
            for j in range(4):
                mj = jnp.full((16,), -1.0, jnp.float32)
                cj = jnp.full((16,), 64, jnp.int32)
                for cc in range(16):
                    c = 16 * j + cc
                    col = plsc.load_gather(
                        u_vm[b], [rows, jnp.full((16,), c, jnp.int32)])
                    gt = col > mj
                    mj = jnp.where(gt, col, mj)
                    cj = jnp.where(gt, c, cj)
                ms.append(mj)
                cs.append(cj)
            m, c1 = ms[0], cs[0]
            for j in range(1, 4):
                win = ms[j] > m
                m = jnp.where(win, ms[j], m)
                c1 = jnp.where(win, cs[j], c1)
            uv_vm[b][pl.ds(g * 16, 16)] = uv_vec
            ut_vm[b][pl.ds(g * 16, 16)] = m
            c1_vm[b][pl.ds(g * 16, 16)] = c1
            lai_vm[b][pl.ds(g * 16, 16)] = la_vec
            l1i_vm[b][pl.ds(g * 16, 16)] = l1_vec
            return carry
        lax.fori_loop(0, _CH // 16, group, 0)

    fire_in(0, base_w)
    fire_in(1, base_w + _CH)

    def pair(k, carry):
        for b in range(2):
            ci = 2 * k + b
            base = base_w + ci * _CH
            wait_in(b, base)

            @pl.when(k > 0)
            def _():
                wait_out(b, base)

            compute_chunk(b)
            fire_out(b, base)

            @pl.when(ci + 2 < nchunks)
            def _():
                fire_in(b, base + 2 * _CH)
        return carry

    lax.fori_loop(0, nchunks // 2, pair, 0)
    wait_out(0, base_w)
    wait_out(1, base_w)

  return _sc_body


def _sc_stage(u, v, batch, time_step, la_pad, l1_pad, off, n):
    f32 = jnp.float32
    i32 = jnp.int32
    mesh = plsc.VectorSubcoreMesh(core_axis_name="c", subcore_axis_name="s")
    out_type = [
        jax.ShapeDtypeStruct((n,), f32),
        jax.ShapeDtypeStruct((n,), f32),
        jax.ShapeDtypeStruct((n,), i32),
        jax.ShapeDtypeStruct((n,), f32),
        jax.ShapeDtypeStruct((n,), f32),
    ]
    scratch = [
        pltpu.VMEM((_CH, 64), f32), pltpu.VMEM((_CH, 64), f32),
        pltpu.VMEM((_CH,), i32), pltpu.VMEM((_CH,), i32),
        pltpu.VMEM((_CH,), i32), pltpu.VMEM((_CH,), i32),
        pltpu.VMEM((64,), i32),
        pltpu.VMEM((128,), f32),
        pltpu.VMEM((128,), f32),
        pltpu.VMEM((_CH,), f32), pltpu.VMEM((_CH,), f32),
        pltpu.VMEM((_CH,), f32), pltpu.VMEM((_CH,), f32),
        pltpu.VMEM((_CH,), i32), pltpu.VMEM((_CH,), i32),
        pltpu.VMEM((_CH,), f32), pltpu.VMEM((_CH,), f32),
        pltpu.VMEM((_CH,), f32), pltpu.VMEM((_CH,), f32),
        pltpu.SemaphoreType.DMA, pltpu.SemaphoreType.DMA,
        pltpu.SemaphoreType.DMA, pltpu.SemaphoreType.DMA,
    ]
    fn = pl.kernel(_make_sc_body(off), mesh=mesh, out_type=out_type,
                   scratch_types=scratch,
                   compiler_params=pltpu.CompilerParams(needs_layout_passes=False))
    return fn(u, v, batch, time_step, la_pad, l1_pad)


def _tc_body(vi_ref, uv_ref, ut_ref, c1_ref, lai_ref, l1i_ref,
             vp_ref, lnvt_ref, lv0_ref):
    BI = vi_ref.shape[0]
    vi = vi_ref[...]
    u_v = uv_ref[...]
    u_top = ut_ref[...]
    c1 = c1_ref[...]
    la = lai_ref[...]
    l1ma = l1i_ref[...]

    z = u_v[0:1, 0:1] * 0.0
    neg30 = jnp.log(z + 1e-30)
    pos = jnp.log(z + 1.0)

    b = l1ma - _LOG_K
    a_hit = pos + la
    m_h = jnp.maximum(a_hit, b)
    q_hit = m_h + jnp.log(jnp.exp(a_hit - m_h) + jnp.exp(b - m_h))
    a_miss = neg30 + la
    m_m = jnp.maximum(a_miss, b)
    q_miss = m_m + jnp.log(jnp.exp(a_miss - m_m) + jnp.exp(b - m_m))

    g_v = -jnp.log(-jnp.log(u_v + 1e-30) + 1e-30)
    g_1 = -jnp.log(-jnp.log(u_top + 1e-30) + 1e-30)
    s_v = g_v + q_hit
    s_1 = g_1 + q_miss

    widx = jnp.where(s_v > s_1, vi,
                     jnp.where(s_1 > s_v, c1, jnp.minimum(vi, c1)))

    iota_c = lax.broadcasted_iota(jnp.int32, (BI, 128, 64), 2)
    eq_w = iota_c == widx[:, :, None]
    oh_v = iota_c == vi[:, :, None]
    pos3 = pos[:, :, None]
    neg3 = neg30[:, :, None]
    vp_ref[...] = eq_w.astype(jnp.float32)
    lnvt_ref[...] = jnp.where(eq_w, pos3, neg3)
    lv0_ref[...] = jnp.where(oh_v, pos3, neg3)


def _tc_stage(vh, sc_outs, C):
    n = vh.shape[0]
    u_v, u_top, c1, la_i, l1_i = sc_outs
    BI = 16
    NR = n // 128
    G = NR // BI
    col = lambda x: x.reshape(NR, 128)
    col_spec = pl.BlockSpec((BI, 128), lambda i: (i, 0))
    out_spec = pl.BlockSpec((BI, 128, C), lambda i: (i, 0, 0))
    out_sds = jax.ShapeDtypeStruct((NR, 128, C), jnp.float32)
    return pl.pallas_call(
        _tc_body,
        grid=(G,),
        in_specs=[col_spec] * 6,
        out_specs=[out_spec, out_spec, out_spec],
        out_shape=[out_sds, out_sds, out_sds],
    )(col(vh), col(u_v), col(u_top), col(c1), col(la_i), col(l1_i))


def kernel(v, time_step, batch, u, log_alphas_bar, log_1_min_alphas_bar):
    N, C = u.shape
    T = log_alphas_bar.shape[0]
    la_pad = jnp.zeros((128,), jnp.float32).at[:T].set(log_alphas_bar)
    l1_pad = jnp.zeros((128,), jnp.float32).at[:T].set(log_1_min_alphas_bar)

    H = N // 2
    sc1 = _sc_stage(u, v, batch, time_step, la_pad, l1_pad, 0, H)
    sc2 = _sc_stage(u, v, batch, time_step, la_pad, l1_pad, H, H)
    vp1, lnvt1, lv01 = _tc_stage(v[:H], sc1, C)
    vp2, lnvt2, lv02 = _tc_stage(v[H:], sc2, C)

    cat = lambda a, b: jnp.concatenate([a, b], axis=0).reshape(N, C)
    return (cat(vp1, vp2), cat(lnvt1, lnvt2), cat(lv01, lv02))


# speedup vs baseline: 1.2695x; 1.2695x over previous
import numpy as np
import jax
import jax.numpy as jnp
from jax import lax
from jax.experimental import pallas as pl
from jax.experimental.pallas import tpu as pltpu
from jax.experimental.pallas import tpu_sc as plsc

_LOG_K = float(np.log(64))
_NW = 32
_CH = 256


def _sc_body(u_hbm, v_hbm, b_hbm, ts_hbm, la_hbm, l1_hbm,
             uv_hbm, ut_hbm, c1_hbm, lai_hbm, l1i_hbm,
             u_vm0, u_vm1, v_vm0, v_vm1, b_vm0, b_vm1,
             ts_vm, la_vm, l1_vm,
             uv_vm0, uv_vm1, ut_vm0, ut_vm1, c1_vm0, c1_vm1,
             lai_vm0, lai_vm1, l1i_vm0, l1i_vm1,
             in_sem0, in_sem1, out_sem0, out_sem1):
    n = u_hbm.shape[0]
    per_w = n // _NW
    nchunks = per_w // _CH
    wid = lax.axis_index("s") * 2 + lax.axis_index("c")
    base_w = wid * per_w

    u_vm = (u_vm0, u_vm1)
    v_vm = (v_vm0, v_vm1)
    b_vm = (b_vm0, b_vm1)
    uv_vm = (uv_vm0, uv_vm1)
    ut_vm = (ut_vm0, ut_vm1)
    c1_vm = (c1_vm0, c1_vm1)
    lai_vm = (lai_vm0, lai_vm1)
    l1i_vm = (l1i_vm0, l1i_vm1)
    in_sem = (in_sem0, in_sem1)
    out_sem = (out_sem0, out_sem1)

    pltpu.sync_copy(ts_hbm, ts_vm)
    pltpu.sync_copy(la_hbm, la_vm)
    pltpu.sync_copy(l1_hbm, l1_vm)

    def fire_in(b, base):
        pltpu.async_copy(u_hbm.at[pl.ds(base, _CH)], u_vm[b], in_sem[b])
        pltpu.async_copy(v_hbm.at[pl.ds(base, _CH)], v_vm[b], in_sem[b])
        pltpu.async_copy(b_hbm.at[pl.ds(base, _CH)], b_vm[b], in_sem[b])

    def wait_in(b, base):
        pltpu.make_async_copy(u_hbm.at[pl.ds(base, _CH)], u_vm[b],
                              in_sem[b]).wait()
        pltpu.make_async_copy(v_hbm.at[pl.ds(base, _CH)], v_vm[b],
                              in_sem[b]).wait()
        pltpu.make_async_copy(b_hbm.at[pl.ds(base, _CH)], b_vm[b],
                              in_sem[b]).wait()

    def fire_out(b, base):
        pltpu.async_copy(uv_vm[b], uv_hbm.at[pl.ds(base, _CH)], out_sem[b])
        pltpu.async_copy(ut_vm[b], ut_hbm.at[pl.ds(base, _CH)], out_sem[b])
        pltpu.async_copy(c1_vm[b], c1_hbm.at[pl.ds(base, _CH)], out_sem[b])
        pltpu.async_copy(lai_vm[b], lai_hbm.at[pl.ds(base, _CH)], out_sem[b])
        pltpu.async_copy(l1i_vm[b], l1i_hbm.at[pl.ds(base, _CH)], out_sem[b])

    def wait_out(b, base):
        pltpu.make_async_copy(uv_vm[b], uv_hbm.at[pl.ds(base, _CH)],
                              out_sem[b]).wait()
        pltpu.make_async_copy(ut_vm[b], ut_hbm.at[pl.ds(base, _CH)],
                              out_sem[b]).wait()
        pltpu.make_async_copy(c1_vm[b], c1_hbm.at[pl.ds(base, _CH)],
                              out_sem[b]).wait()
        pltpu.make_async_copy(lai_vm[b], lai_hbm.at[pl.ds(base, _CH)],
                              out_sem[b]).wait()
        pltpu.make_async_copy(l1i_vm[b], l1i_hbm.at[pl.ds(base, _CH)],
                              out_sem[b]).wait()

    def compute_chunk(b):
        def group(g, carry):
            rows = lax.iota(jnp.int32, 16) + (g * 16)
            v_vec = v_vm[b][pl.ds(g * 16, 16)]
            b_vec = b_vm[b][pl.ds(g * 16, 16)]
            t_vec = plsc.load_gather(ts_vm, [b_vec])
            la_vec = plsc.load_gather(la_vm, [t_vec])
            l1_vec = plsc.load_gather(l1_vm, [t_vec])
            uv_vec = plsc.load_gather(u_vm[b], [rows, v_vec])
            plsc.store_scatter(u_vm[b], [rows, v_vec],
                               jnp.full((16,), -1.0, jnp.float32))
            ms, cs = [], []
            for j in range(4):
                mj = jnp.full((16,), -1.0, jnp.float32)
                cj = jnp.full((16,), 64, jnp.int32)
                for cc in range(16):
                    c = 16 * j + cc
                    col = plsc.load_gather(
                        u_vm[b], [rows, jnp.full((16,), c, jnp.int32)])
                    gt = col > mj
                    mj = jnp.where(gt, col, mj)
                    cj = jnp.where(gt, c, cj)
                ms.append(mj)
                cs.append(cj)
            m, c1 = ms[0], cs[0]
            for j in range(1, 4):
                win = ms[j] > m
                m = jnp.where(win, ms[j], m)
                c1 = jnp.where(win, cs[j], c1)
            uv_vm[b][pl.ds(g * 16, 16)] = uv_vec
            ut_vm[b][pl.ds(g * 16, 16)] = m
            c1_vm[b][pl.ds(g * 16, 16)] = c1
            lai_vm[b][pl.ds(g * 16, 16)] = la_vec
            l1i_vm[b][pl.ds(g * 16, 16)] = l1_vec
            return carry
        lax.fori_loop(0, _CH // 16, group, 0)

    fire_in(0, base_w)
    fire_in(1, base_w + _CH)

    def pair(k, carry):
        for b in range(2):
            ci = 2 * k + b
            base = base_w + ci * _CH
            wait_in(b, base)

            @pl.when(k > 0)
            def _():
                wait_out(b, base)

            compute_chunk(b)
            fire_out(b, base)

            @pl.when(ci + 2 < nchunks)
            def _():
                fire_in(b, base + 2 * _CH)
        return carry

    lax.fori_loop(0, nchunks // 2, pair, 0)
    wait_out(0, base_w)
    wait_out(1, base_w)


def _sc_stage(u, v, batch, time_step, la_pad, l1_pad):
    n = u.shape[0]
    f32 = jnp.float32
    i32 = jnp.int32
    mesh = plsc.VectorSubcoreMesh(core_axis_name="c", subcore_axis_name="s")
    out_type = [
        jax.ShapeDtypeStruct((n,), f32),
        jax.ShapeDtypeStruct((n,), f32),
        jax.ShapeDtypeStruct((n,), i32),
        jax.ShapeDtypeStruct((n,), f32),
        jax.ShapeDtypeStruct((n,), f32),
    ]
    scratch = [
        pltpu.VMEM((_CH, 64), f32), pltpu.VMEM((_CH, 64), f32),
        pltpu.VMEM((_CH,), i32), pltpu.VMEM((_CH,), i32),
        pltpu.VMEM((_CH,), i32), pltpu.VMEM((_CH,), i32),
        pltpu.VMEM((64,), i32),
        pltpu.VMEM((128,), f32),
        pltpu.VMEM((128,), f32),
        pltpu.VMEM((_CH,), f32), pltpu.VMEM((_CH,), f32),
        pltpu.VMEM((_CH,), f32), pltpu.VMEM((_CH,), f32),
        pltpu.VMEM((_CH,), i32), pltpu.VMEM((_CH,), i32),
        pltpu.VMEM((_CH,), f32), pltpu.VMEM((_CH,), f32),
        pltpu.VMEM((_CH,), f32), pltpu.VMEM((_CH,), f32),
        pltpu.SemaphoreType.DMA, pltpu.SemaphoreType.DMA,
        pltpu.SemaphoreType.DMA, pltpu.SemaphoreType.DMA,
    ]
    fn = pl.kernel(_sc_body, mesh=mesh, out_type=out_type, scratch_types=scratch,
                   compiler_params=pltpu.CompilerParams(needs_layout_passes=False))
    return fn(u, v, batch, time_step, la_pad, l1_pad)


def _tc_body(vi_ref, uv_ref, ut_ref, c1_ref, lai_ref, l1i_ref,
             vp_ref, lnvt_ref, lv0_ref):
    BI = vi_ref.shape[0]
    vi = vi_ref[...]
    u_v = uv_ref[...]
    u_top = ut_ref[...]
    c1 = c1_ref[...]
    la = lai_ref[...]
    l1ma = l1i_ref[...]

    z = u_v[0:1, 0:1] * 0.0
    neg30 = jnp.log(z + 1e-30)
    pos = jnp.log(z + 1.0)

    b = l1ma - _LOG_K
    a_hit = pos + la
    m_h = jnp.maximum(a_hit, b)
    q_hit = m_h + jnp.log(jnp.exp(a_hit - m_h) + jnp.exp(b - m_h))
    a_miss = neg30 + la
    m_m = jnp.maximum(a_miss, b)
    q_miss = m_m + jnp.log(jnp.exp(a_miss - m_m) + jnp.exp(b - m_m))

    g_v = -jnp.log(-jnp.log(u_v + 1e-30) + 1e-30)
    g_1 = -jnp.log(-jnp.log(u_top + 1e-30) + 1e-30)
    s_v = g_v + q_hit
    s_1 = g_1 + q_miss

    widx = jnp.where(s_v > s_1, vi,
                     jnp.where(s_1 > s_v, c1, jnp.minimum(vi, c1)))

    iota_c = lax.broadcasted_iota(jnp.int32, (BI, 128, 64), 2)
    eq_w = iota_c == widx[:, :, None]
    oh_v = iota_c == vi[:, :, None]
    pos3 = pos[:, :, None]
    neg3 = neg30[:, :, None]
    vp_ref[...] = eq_w.astype(jnp.float32)
    lnvt_ref[...] = jnp.where(eq_w, pos3, neg3)
    lv0_ref[...] = jnp.where(oh_v, pos3, neg3)


def kernel(v, time_step, batch, u, log_alphas_bar, log_1_min_alphas_bar):
    N, C = u.shape
    T = log_alphas_bar.shape[0]
    la_pad = jnp.zeros((128,), jnp.float32).at[:T].set(log_alphas_bar)
    l1_pad = jnp.zeros((128,), jnp.float32).at[:T].set(log_1_min_alphas_bar)

    u_v, u_top, c1, la_i, l1_i = _sc_stage(u, v, batch, time_step,
                                           la_pad, l1_pad)

    BI = 16
    NR = N // 128
    G = NR // BI
    col = lambda x: x.reshape(NR, 128)
    col_spec = pl.BlockSpec((BI, 128), lambda i: (i, 0))
    out_spec = pl.BlockSpec((BI, 128, C), lambda i: (i, 0, 0))
    out_sds = jax.ShapeDtypeStruct((NR, 128, C), jnp.float32)

    vp, lnvt, lv0 = pl.pallas_call(
        _tc_body,
        grid=(G,),
        in_specs=[col_spec] * 6,
        out_specs=[out_spec, out_spec, out_spec],
        out_shape=[out_sds, out_sds, out_sds],
    )(col(v), col(u_v), col(u_top), col(c1), col(la_i), col(l1_i))
    return (vp.reshape(N, C), lnvt.reshape(N, C), lv0.reshape(N, C))


# speedup vs baseline: 1.3118x; 1.0333x over previous
import numpy as np
import jax
import jax.numpy as jnp
from jax import lax
from jax.experimental import pallas as pl
from jax.experimental.pallas import tpu as pltpu
from jax.experimental.pallas import tpu_sc as plsc

_LOG_K = float(np.log(64))
_NW = 32
_CH = 256


def _sc_body(u_hbm, v_hbm, b_hbm, ts_hbm, la_hbm, l1_hbm,
             uv_hbm, ut_hbm, c1_hbm, lai_hbm, l1i_hbm,
             u_vm0, u_vm1, v_vm0, v_vm1, b_vm0, b_vm1,
             ts_vm, la_vm, l1_vm,
             uv_vm0, uv_vm1, ut_vm0, ut_vm1, c1_vm0, c1_vm1,
             lai_vm0, lai_vm1, l1i_vm0, l1i_vm1,
             in_sem0, in_sem1, out_sem0, out_sem1):
    n = u_hbm.shape[0]
    per_w = n // _NW
    nchunks = per_w // _CH
    wid = lax.axis_index("s") * 2 + lax.axis_index("c")
    base_w = wid * per_w

    u_vm = (u_vm0, u_vm1)
    v_vm = (v_vm0, v_vm1)
    b_vm = (b_vm0, b_vm1)
    uv_vm = (uv_vm0, uv_vm1)
    ut_vm = (ut_vm0, ut_vm1)
    c1_vm = (c1_vm0, c1_vm1)
    lai_vm = (lai_vm0, lai_vm1)
    l1i_vm = (l1i_vm0, l1i_vm1)
    in_sem = (in_sem0, in_sem1)
    out_sem = (out_sem0, out_sem1)

    pltpu.sync_copy(ts_hbm, ts_vm)
    pltpu.sync_copy(la_hbm, la_vm)
    pltpu.sync_copy(l1_hbm, l1_vm)

    def fire_in(b, base):
        pltpu.async_copy(u_hbm.at[pl.ds(base, _CH)], u_vm[b], in_sem[b])
        pltpu.async_copy(v_hbm.at[pl.ds(base, _CH)], v_vm[b], in_sem[b])
        pltpu.async_copy(b_hbm.at[pl.ds(base, _CH)], b_vm[b], in_sem[b])

    def wait_in(b, base):
        pltpu.make_async_copy(u_hbm.at[pl.ds(base, _CH)], u_vm[b],
                              in_sem[b]).wait()
        pltpu.make_async_copy(v_hbm.at[pl.ds(base, _CH)], v_vm[b],
                              in_sem[b]).wait()
        pltpu.make_async_copy(b_hbm.at[pl.ds(base, _CH)], b_vm[b],
                              in_sem[b]).wait()

    def fire_out(b, base):
        pltpu.async_copy(uv_vm[b], uv_hbm.at[pl.ds(base, _CH)], out_sem[b])
        pltpu.async_copy(ut_vm[b], ut_hbm.at[pl.ds(base, _CH)], out_sem[b])
        pltpu.async_copy(c1_vm[b], c1_hbm.at[pl.ds(base, _CH)], out_sem[b])
        pltpu.async_copy(lai_vm[b], lai_hbm.at[pl.ds(base, _CH)], out_sem[b])
        pltpu.async_copy(l1i_vm[b], l1i_hbm.at[pl.ds(base, _CH)], out_sem[b])

    def wait_out(b, base):
        pltpu.make_async_copy(uv_vm[b], uv_hbm.at[pl.ds(base, _CH)],
                              out_sem[b]).wait()
        pltpu.make_async_copy(ut_vm[b], ut_hbm.at[pl.ds(base, _CH)],
                              out_sem[b]).wait()
        pltpu.make_async_copy(c1_vm[b], c1_hbm.at[pl.ds(base, _CH)],
                              out_sem[b]).wait()
        pltpu.make_async_copy(lai_vm[b], lai_hbm.at[pl.ds(base, _CH)],
                              out_sem[b]).wait()
        pltpu.make_async_copy(l1i_vm[b], l1i_hbm.at[pl.ds(base, _CH)],
                              out_sem[b]).wait()

    def compute_chunk(b):
        def group(g, carry):
            rows = lax.iota(jnp.int32, 16) + (g * 16)
            v_vec = v_vm[b][pl.ds(g * 16, 16)]
            b_vec = b_vm[b][pl.ds(g * 16, 16)]
            t_vec = plsc.load_gather(ts_vm, [b_vec])
            la_vec = plsc.load_gather(la_vm, [t_vec])
            l1_vec = plsc.load_gather(l1_vm, [t_vec])
            uv_vec = plsc.load_gather(u_vm[b], [rows, v_vec])
            plsc.store_scatter(u_vm[b], [rows, v_vec],
                               jnp.full((16,), -1.0, jnp.float32))
            ms, cs = [], []
            for j in range(4):
                mj = jnp.full((16,), -1.0, jnp.float32)
                cj = jnp.full((16,), 64, jnp.int32)
                for cc in range(16):
                    c = 16 * j + cc
                    col = plsc.load_gather(
                        u_vm[b], [rows, jnp.full((16,), c, jnp.int32)])
                    gt = col > mj
                    mj = jnp.where(gt, col, mj)
                    cj = jnp.where(gt, c, cj)
                ms.append(mj)
                cs.append(cj)
            m, c1 = ms[0], cs[0]
            for j in range(1, 4):
                win = ms[j] > m
                m = jnp.where(win, ms[j], m)
                c1 = jnp.where(win, cs[j], c1)
            uv_vm[b][pl.ds(g * 16, 16)] = uv_vec
            ut_vm[b][pl.ds(g * 16, 16)] = m
            c1_vm[b][pl.ds(g * 16, 16)] = c1
            lai_vm[b][pl.ds(g * 16, 16)] = la_vec
            l1i_vm[b][pl.ds(g * 16, 16)] = l1_vec
            return carry
        lax.fori_loop(0, _CH // 16, group, 0)

    fire_in(0, base_w)
    fire_in(1, base_w + _CH)

    def pair(k, carry):
        for b in range(2):
            ci = 2 * k + b
            base = base_w + ci * _CH
            wait_in(b, base)

            @pl.when(k > 0)
            def _():
                wait_out(b, base)

            compute_chunk(b)
            fire_out(b, base)

            @pl.when(ci + 2 < nchunks)
            def _():
                fire_in(b, base + 2 * _CH)
        return carry

    lax.fori_loop(0, nchunks // 2, pair, 0)
    wait_out(0, base_w)
    wait_out(1, base_w)


def _sc_stage(u, v, batch, time_step, la_pad, l1_pad):
    n = u.shape[0]
    f32 = jnp.float32
    i32 = jnp.int32
    mesh = plsc.VectorSubcoreMesh(core_axis_name="c", subcore_axis_name="s")
    out_type = [
        jax.ShapeDtypeStruct((n,), f32),
        jax.ShapeDtypeStruct((n,), f32),
        jax.ShapeDtypeStruct((n,), i32),
        jax.ShapeDtypeStruct((n,), f32),
        jax.ShapeDtypeStruct((n,), f32),
    ]
    scratch = [
        pltpu.VMEM((_CH, 64), f32), pltpu.VMEM((_CH, 64), f32),
        pltpu.VMEM((_CH,), i32), pltpu.VMEM((_CH,), i32),
        pltpu.VMEM((_CH,), i32), pltpu.VMEM((_CH,), i32),
        pltpu.VMEM((64,), i32),
        pltpu.VMEM((128,), f32),
        pltpu.VMEM((128,), f32),
        pltpu.VMEM((_CH,), f32), pltpu.VMEM((_CH,), f32),
        pltpu.VMEM((_CH,), f32), pltpu.VMEM((_CH,), f32),
        pltpu.VMEM((_CH,), i32), pltpu.VMEM((_CH,), i32),
        pltpu.VMEM((_CH,), f32), pltpu.VMEM((_CH,), f32),
        pltpu.VMEM((_CH,), f32), pltpu.VMEM((_CH,), f32),
        pltpu.SemaphoreType.DMA, pltpu.SemaphoreType.DMA,
        pltpu.SemaphoreType.DMA, pltpu.SemaphoreType.DMA,
    ]
    fn = pl.kernel(_sc_body, mesh=mesh, out_type=out_type, scratch_types=scratch,
                   compiler_params=pltpu.CompilerParams(needs_layout_passes=False))
    return fn(u, v, batch, time_step, la_pad, l1_pad)


def _tc_body(vi_ref, uv_ref, ut_ref, c1_ref, lai_ref, l1i_ref,
             vp_ref, lnvt_ref, lv0_ref):
    BI = vi_ref.shape[0]
    vi = vi_ref[...]
    u_v = uv_ref[...]
    u_top = ut_ref[...]
    c1 = c1_ref[...]
    la = lai_ref[...]
    l1ma = l1i_ref[...]

    z = u_v[0:1, 0:1] * 0.0
    neg30 = jnp.log(z + 1e-30)
    pos = jnp.log(z + 1.0)

    b = l1ma - _LOG_K
    a_hit = pos + la
    m_h = jnp.maximum(a_hit, b)
    q_hit = m_h + jnp.log(jnp.exp(a_hit - m_h) + jnp.exp(b - m_h))
    a_miss = neg30 + la
    m_m = jnp.maximum(a_miss, b)
    q_miss = m_m + jnp.log(jnp.exp(a_miss - m_m) + jnp.exp(b - m_m))

    g_v = -jnp.log(-jnp.log(u_v + 1e-30) + 1e-30)
    g_1 = -jnp.log(-jnp.log(u_top + 1e-30) + 1e-30)
    s_v = g_v + q_hit
    s_1 = g_1 + q_miss

    widx = jnp.where(s_v > s_1, vi,
                     jnp.where(s_1 > s_v, c1, jnp.minimum(vi, c1)))

    iota_c = lax.broadcasted_iota(jnp.int32, (BI, 128, 64), 2)
    eq_w = iota_c == widx[:, :, None]
    oh_v = iota_c == vi[:, :, None]
    pos3 = pos[:, :, None]
    neg3 = neg30[:, :, None]
    vp_ref[...] = eq_w.astype(jnp.float32)
    lnvt_ref[...] = jnp.where(eq_w, pos3, neg3)
    lv0_ref[...] = jnp.where(oh_v, pos3, neg3)


def kernel(v, time_step, batch, u, log_alphas_bar, log_1_min_alphas_bar):
    N, C = u.shape
    T = log_alphas_bar.shape[0]
    la_pad = jnp.zeros((128,), jnp.float32).at[:T].set(log_alphas_bar)
    l1_pad = jnp.zeros((128,), jnp.float32).at[:T].set(log_1_min_alphas_bar)

    u_v, u_top, c1, la_i, l1_i = _sc_stage(u, v, batch, time_step,
                                           la_pad, l1_pad)

    BI = 32
    NR = N // 128
    G = NR // BI
    col = lambda x: x.reshape(NR, 128)
    col_spec = pl.BlockSpec((BI, 128), lambda i: (i, 0))
    out_spec = pl.BlockSpec((BI, 128, C), lambda i: (i, 0, 0))
    out_sds = jax.ShapeDtypeStruct((NR, 128, C), jnp.float32)

    vp, lnvt, lv0 = pl.pallas_call(
        _tc_body,
        grid=(G,),
        in_specs=[col_spec] * 6,
        out_specs=[out_spec, out_spec, out_spec],
        out_shape=[out_sds, out_sds, out_sds],
    )(col(v), col(u_v), col(u_top), col(c1), col(la_i), col(l1_i))
    return (vp.reshape(N, C), lnvt.reshape(N, C), lv0.reshape(N, C))


# speedup vs baseline: 1.3118x; 1.0000x over previous
import numpy as np
import jax
import jax.numpy as jnp
from jax import lax
from jax.experimental import pallas as pl
from jax.experimental.pallas import tpu as pltpu
from jax.experimental.pallas import tpu_sc as plsc

_LOG_K = float(np.log(64))
_NW = 32
_CH = 256


def _sc_body(u_hbm, v_hbm, b_hbm, ts_hbm, la_hbm, l1_hbm,
             uv_hbm, ut_hbm, c1_hbm, lai_hbm, l1i_hbm,
             u_vm0, u_vm1, v_vm0, v_vm1, b_vm0, b_vm1,
             ts_vm, la_vm, l1_vm,
             uv_vm0, uv_vm1, ut_vm0, ut_vm1, c1_vm0, c1_vm1,
             lai_vm0, lai_vm1, l1i_vm0, l1i_vm1,
             in_sem0, in_sem1, out_sem0, out_sem1):
    n = u_hbm.shape[0]
    per_w = n // _NW
    nchunks = per_w // _CH
    wid = lax.axis_index("s") * 2 + lax.axis_index("c")
    base_w = wid * per_w

    u_vm = (u_vm0, u_vm1)
    v_vm = (v_vm0, v_vm1)
    b_vm = (b_vm0, b_vm1)
    uv_vm = (uv_vm0, uv_vm1)
    ut_vm = (ut_vm0, ut_vm1)
    c1_vm = (c1_vm0, c1_vm1)
    lai_vm = (lai_vm0, lai_vm1)
    l1i_vm = (l1i_vm0, l1i_vm1)
    in_sem = (in_sem0, in_sem1)
    out_sem = (out_sem0, out_sem1)

    pltpu.sync_copy(ts_hbm, ts_vm)
    pltpu.sync_copy(la_hbm, la_vm)
    pltpu.sync_copy(l1_hbm, l1_vm)

    def fire_in(b, base):
        pltpu.async_copy(u_hbm.at[pl.ds(base, _CH)], u_vm[b], in_sem[b])
        pltpu.async_copy(v_hbm.at[pl.ds(base, _CH)], v_vm[b], in_sem[b])
        pltpu.async_copy(b_hbm.at[pl.ds(base, _CH)], b_vm[b], in_sem[b])

    def wait_in(b, base):
        pltpu.make_async_copy(u_hbm.at[pl.ds(base, _CH)], u_vm[b],
                              in_sem[b]).wait()
        pltpu.make_async_copy(v_hbm.at[pl.ds(base, _CH)], v_vm[b],
                              in_sem[b]).wait()
        pltpu.make_async_copy(b_hbm.at[pl.ds(base, _CH)], b_vm[b],
                              in_sem[b]).wait()

    def fire_out(b, base):
        pltpu.async_copy(uv_vm[b], uv_hbm.at[pl.ds(base, _CH)], out_sem[b])
        pltpu.async_copy(ut_vm[b], ut_hbm.at[pl.ds(base, _CH)], out_sem[b])
        pltpu.async_copy(c1_vm[b], c1_hbm.at[pl.ds(base, _CH)], out_sem[b])
        pltpu.async_copy(lai_vm[b], lai_hbm.at[pl.ds(base, _CH)], out_sem[b])
        pltpu.async_copy(l1i_vm[b], l1i_hbm.at[pl.ds(base, _CH)], out_sem[b])

    def wait_out(b, base):
        pltpu.make_async_copy(uv_vm[b], uv_hbm.at[pl.ds(base, _CH)],
                              out_sem[b]).wait()
        pltpu.make_async_copy(ut_vm[b], ut_hbm.at[pl.ds(base, _CH)],
                              out_sem[b]).wait()
        pltpu.make_async_copy(c1_vm[b], c1_hbm.at[pl.ds(base, _CH)],
                              out_sem[b]).wait()
        pltpu.make_async_copy(lai_vm[b], lai_hbm.at[pl.ds(base, _CH)],
                              out_sem[b]).wait()
        pltpu.make_async_copy(l1i_vm[b], l1i_hbm.at[pl.ds(base, _CH)],
                              out_sem[b]).wait()

    def compute_chunk(b):
        def group(g, carry):
            rows = lax.iota(jnp.int32, 16) + (g * 16)
            v_vec = v_vm[b][pl.ds(g * 16, 16)]
            b_vec = b_vm[b][pl.ds(g * 16, 16)]
            t_vec = plsc.load_gather(ts_vm, [b_vec])
            la_vec = plsc.load_gather(la_vm, [t_vec])
            l1_vec = plsc.load_gather(l1_vm, [t_vec])
            uv_vec = plsc.load_gather(u_vm[b], [rows, v_vec])
            plsc.store_scatter(u_vm[b], [rows, v_vec],
                               jnp.full((16,), -1.0, jnp.float32))
            ms, cs = [], []
            for j in range(4):
                mj = jnp.full((16,), -1.0, jnp.float32)
                cj = jnp.full((16,), 64, jnp.int32)
                for cc in range(16):
                    c = 16 * j + cc
                    col = plsc.load_gather(
                        u_vm[b], [rows, jnp.full((16,), c, jnp.int32)])
                    gt = col > mj
                    mj = jnp.where(gt, col, mj)
                    cj = jnp.where(gt, c, cj)
                ms.append(mj)
                cs.append(cj)
            m, c1 = ms[0], cs[0]
            for j in range(1, 4):
                win = ms[j] > m
                m = jnp.where(win, ms[j], m)
                c1 = jnp.where(win, cs[j], c1)
            uv_vm[b][pl.ds(g * 16, 16)] = uv_vec
            ut_vm[b][pl.ds(g * 16, 16)] = m
            c1_vm[b][pl.ds(g * 16, 16)] = c1
            lai_vm[b][pl.ds(g * 16, 16)] = la_vec
            l1i_vm[b][pl.ds(g * 16, 16)] = l1_vec
            return carry
        lax.fori_loop(0, _CH // 16, group, 0)

    fire_in(0, base_w)
    fire_in(1, base_w + _CH)

    def pair(k, carry):
        for b in range(2):
            ci = 2 * k + b
            base = base_w + ci * _CH
            wait_in(b, base)

            @pl.when(k > 0)
            def _():
                wait_out(b, base)

            compute_chunk(b)
            fire_out(b, base)

            @pl.when(ci + 2 < nchunks)
            def _():
                fire_in(b, base + 2 * _CH)
        return carry

    lax.fori_loop(0, nchunks // 2, pair, 0)
    wait_out(0, base_w)
    wait_out(1, base_w)


def _sc_stage(u, v, batch, time_step, la_pad, l1_pad):
    n = u.shape[0]
    f32 = jnp.float32
    i32 = jnp.int32
    mesh = plsc.VectorSubcoreMesh(core_axis_name="c", subcore_axis_name="s")
    out_type = [
        jax.ShapeDtypeStruct((n,), f32),
        jax.ShapeDtypeStruct((n,), f32),
        jax.ShapeDtypeStruct((n,), i32),
        jax.ShapeDtypeStruct((n,), f32),
        jax.ShapeDtypeStruct((n,), f32),
    ]
    scratch = [
        pltpu.VMEM((_CH, 64), f32), pltpu.VMEM((_CH, 64), f32),
        pltpu.VMEM((_CH,), i32), pltpu.VMEM((_CH,), i32),
        pltpu.VMEM((_CH,), i32), pltpu.VMEM((_CH,), i32),
        pltpu.VMEM((64,), i32),
        pltpu.VMEM((128,), f32),
        pltpu.VMEM((128,), f32),
        pltpu.VMEM((_CH,), f32), pltpu.VMEM((_CH,), f32),
        pltpu.VMEM((_CH,), f32), pltpu.VMEM((_CH,), f32),
        pltpu.VMEM((_CH,), i32), pltpu.VMEM((_CH,), i32),
        pltpu.VMEM((_CH,), f32), pltpu.VMEM((_CH,), f32),
        pltpu.VMEM((_CH,), f32), pltpu.VMEM((_CH,), f32),
        pltpu.SemaphoreType.DMA, pltpu.SemaphoreType.DMA,
        pltpu.SemaphoreType.DMA, pltpu.SemaphoreType.DMA,
    ]
    fn = pl.kernel(_sc_body, mesh=mesh, out_type=out_type, scratch_types=scratch,
                   compiler_params=pltpu.CompilerParams(needs_layout_passes=False))
    return fn(u, v, batch, time_step, la_pad, l1_pad)


def _tc_body(vi_ref, uv_ref, ut_ref, c1_ref, lai_ref, l1i_ref,
             vp_ref, lnvt_ref, lv0_ref):
    BI = vi_ref.shape[0]
    vi = vi_ref[...]
    u_v = uv_ref[...]
    u_top = ut_ref[...]
    c1 = c1_ref[...]
    la = lai_ref[...]
    l1ma = l1i_ref[...]

    z = u_v[0:1, 0:1] * 0.0
    neg30 = jnp.log(z + 1e-30)
    pos = jnp.log(z + 1.0)

    b = l1ma - _LOG_K
    a_hit = pos + la
    m_h = jnp.maximum(a_hit, b)
    q_hit = m_h + jnp.log(jnp.exp(a_hit - m_h) + jnp.exp(b - m_h))
    a_miss = neg30 + la
    m_m = jnp.maximum(a_miss, b)
    q_miss = m_m + jnp.log(jnp.exp(a_miss - m_m) + jnp.exp(b - m_m))

    g_v = -jnp.log(-jnp.log(u_v + 1e-30) + 1e-30)
    g_1 = -jnp.log(-jnp.log(u_top + 1e-30) + 1e-30)
    s_v = g_v + q_hit
    s_1 = g_1 + q_miss

    widx = jnp.where(s_v > s_1, vi,
                     jnp.where(s_1 > s_v, c1, jnp.minimum(vi, c1)))

    iota_c = lax.broadcasted_iota(jnp.int32, (BI, 128, 64), 2)
    eq_w = iota_c == widx[:, :, None]
    oh_v = iota_c == vi[:, :, None]
    pos3 = pos[:, :, None]
    neg3 = neg30[:, :, None]
    vp_ref[...] = eq_w.astype(jnp.float32)
    lnvt_ref[...] = jnp.where(eq_w, pos3, neg3)
    lv0_ref[...] = jnp.where(oh_v, pos3, neg3)


def kernel(v, time_step, batch, u, log_alphas_bar, log_1_min_alphas_bar):
    N, C = u.shape
    T = log_alphas_bar.shape[0]
    la_pad = jnp.zeros((128,), jnp.float32).at[:T].set(log_alphas_bar)
    l1_pad = jnp.zeros((128,), jnp.float32).at[:T].set(log_1_min_alphas_bar)

    u_v, u_top, c1, la_i, l1_i = _sc_stage(u, v, batch, time_step,
                                           la_pad, l1_pad)

    BI = 64
    NR = N // 128
    G = NR // BI
    col = lambda x: x.reshape(NR, 128)
    col_spec = pl.BlockSpec((BI, 128), lambda i: (i, 0))
    out_spec = pl.BlockSpec((BI, 128, C), lambda i: (i, 0, 0))
    out_sds = jax.ShapeDtypeStruct((NR, 128, C), jnp.float32)

    vp, lnvt, lv0 = pl.pallas_call(
        _tc_body,
        grid=(G,),
        in_specs=[col_spec] * 6,
        out_specs=[out_spec, out_spec, out_spec],
        out_shape=[out_sds, out_sds, out_sds],
    )(col(v), col(u_v), col(u_top), col(c1), col(la_i), col(l1_i))
    return (vp.reshape(N, C), lnvt.reshape(N, C), lv0.reshape(N, C))


# speedup vs baseline: 1.3167x; 1.0037x over previous
import numpy as np
import jax
import jax.numpy as jnp
from jax import lax
from jax.experimental import pallas as pl
from jax.experimental.pallas import tpu as pltpu
from jax.experimental.pallas import tpu_sc as plsc

_LOG_K = float(np.log(64))
_NW = 32
_CH = 128


def _sc_body(u_hbm, v_hbm, b_hbm, ts_hbm, la_hbm, l1_hbm,
             uv_hbm, ut_hbm, c1_hbm, lai_hbm, l1i_hbm,
             u_vm0, u_vm1, v_vm0, v_vm1, b_vm0, b_vm1,
             ts_vm, la_vm, l1_vm,
             uv_vm0, uv_vm1, ut_vm0, ut_vm1, c1_vm0, c1_vm1,
             lai_vm0, lai_vm1, l1i_vm0, l1i_vm1,
             in_sem0, in_sem1, out_sem0, out_sem1):
    n = u_hbm.shape[0]
    per_w = n // _NW
    nchunks = per_w // _CH
    wid = lax.axis_index("s") * 2 + lax.axis_index("c")
    base_w = wid * per_w

    u_vm = (u_vm0, u_vm1)
    v_vm = (v_vm0, v_vm1)
    b_vm = (b_vm0, b_vm1)
    uv_vm = (uv_vm0, uv_vm1)
    ut_vm = (ut_vm0, ut_vm1)
    c1_vm = (c1_vm0, c1_vm1)
    lai_vm = (lai_vm0, lai_vm1)
    l1i_vm = (l1i_vm0, l1i_vm1)
    in_sem = (in_sem0, in_sem1)
    out_sem = (out_sem0, out_sem1)

    pltpu.sync_copy(ts_hbm, ts_vm)
    pltpu.sync_copy(la_hbm, la_vm)
    pltpu.sync_copy(l1_hbm, l1_vm)

    def fire_in(b, base):
        pltpu.async_copy(u_hbm.at[pl.ds(base, _CH)], u_vm[b], in_sem[b])
        pltpu.async_copy(v_hbm.at[pl.ds(base, _CH)], v_vm[b], in_sem[b])
        pltpu.async_copy(b_hbm.at[pl.ds(base, _CH)], b_vm[b], in_sem[b])

    def wait_in(b, base):
        pltpu.make_async_copy(u_hbm.at[pl.ds(base, _CH)], u_vm[b],
                              in_sem[b]).wait()
        pltpu.make_async_copy(v_hbm.at[pl.ds(base, _CH)], v_vm[b],
                              in_sem[b]).wait()
        pltpu.make_async_copy(b_hbm.at[pl.ds(base, _CH)], b_vm[b],
                              in_sem[b]).wait()

    def fire_out(b, base):
        pltpu.async_copy(uv_vm[b], uv_hbm.at[pl.ds(base, _CH)], out_sem[b])
        pltpu.async_copy(ut_vm[b], ut_hbm.at[pl.ds(base, _CH)], out_sem[b])
        pltpu.async_copy(c1_vm[b], c1_hbm.at[pl.ds(base, _CH)], out_sem[b])
        pltpu.async_copy(lai_vm[b], lai_hbm.at[pl.ds(base, _CH)], out_sem[b])
        pltpu.async_copy(l1i_vm[b], l1i_hbm.at[pl.ds(base, _CH)], out_sem[b])

    def wait_out(b, base):
        pltpu.make_async_copy(uv_vm[b], uv_hbm.at[pl.ds(base, _CH)],
                              out_sem[b]).wait()
        pltpu.make_async_copy(ut_vm[b], ut_hbm.at[pl.ds(base, _CH)],
                              out_sem[b]).wait()
        pltpu.make_async_copy(c1_vm[b], c1_hbm.at[pl.ds(base, _CH)],
                              out_sem[b]).wait()
        pltpu.make_async_copy(lai_vm[b], lai_hbm.at[pl.ds(base, _CH)],
                              out_sem[b]).wait()
        pltpu.make_async_copy(l1i_vm[b], l1i_hbm.at[pl.ds(base, _CH)],
                              out_sem[b]).wait()

    def compute_chunk(b):
        def group(g, carry):
            rows = lax.iota(jnp.int32, 16) + (g * 16)
            v_vec = v_vm[b][pl.ds(g * 16, 16)]
            b_vec = b_vm[b][pl.ds(g * 16, 16)]
            t_vec = plsc.load_gather(ts_vm, [b_vec])
            la_vec = plsc.load_gather(la_vm, [t_vec])
            l1_vec = plsc.load_gather(l1_vm, [t_vec])
            uv_vec = plsc.load_gather(u_vm[b], [rows, v_vec])
            plsc.store_scatter(u_vm[b], [rows, v_vec],
                               jnp.full((16,), -1.0, jnp.float32))
            ms, cs = [], []
            for j in range(4):
                mj = jnp.full((16,), -1.0, jnp.float32)
                cj = jnp.full((16,), 64, jnp.int32)
                for cc in range(16):
                    c = 16 * j + cc
                    col = plsc.load_gather(
                        u_vm[b], [rows, jnp.full((16,), c, jnp.int32)])
                    gt = col > mj
                    mj = jnp.where(gt, col, mj)
                    cj = jnp.where(gt, c, cj)
                ms.append(mj)
                cs.append(cj)
            m, c1 = ms[0], cs[0]
            for j in range(1, 4):
                win = ms[j] > m
                m = jnp.where(win, ms[j], m)
                c1 = jnp.where(win, cs[j], c1)
            uv_vm[b][pl.ds(g * 16, 16)] = uv_vec
            ut_vm[b][pl.ds(g * 16, 16)] = m
            c1_vm[b][pl.ds(g * 16, 16)] = c1
            lai_vm[b][pl.ds(g * 16, 16)] = la_vec
            l1i_vm[b][pl.ds(g * 16, 16)] = l1_vec
            return carry
        lax.fori_loop(0, _CH // 16, group, 0)

    fire_in(0, base_w)
    fire_in(1, base_w + _CH)

    def pair(k, carry):
        for b in range(2):
            ci = 2 * k + b
            base = base_w + ci * _CH
            wait_in(b, base)

            @pl.when(k > 0)
            def _():
                wait_out(b, base)

            compute_chunk(b)
            fire_out(b, base)

            @pl.when(ci + 2 < nchunks)
            def _():
                fire_in(b, base + 2 * _CH)
        return carry

    lax.fori_loop(0, nchunks // 2, pair, 0)
    wait_out(0, base_w)
    wait_out(1, base_w)


def _sc_stage(u, v, batch, time_step, la_pad, l1_pad):
    n = u.shape[0]
    f32 = jnp.float32
    i32 = jnp.int32
    mesh = plsc.VectorSubcoreMesh(core_axis_name="c", subcore_axis_name="s")
    out_type = [
        jax.ShapeDtypeStruct((n,), f32),
        jax.ShapeDtypeStruct((n,), f32),
        jax.ShapeDtypeStruct((n,), i32),
        jax.ShapeDtypeStruct((n,), f32),
        jax.ShapeDtypeStruct((n,), f32),
    ]
    scratch = [
        pltpu.VMEM((_CH, 64), f32), pltpu.VMEM((_CH, 64), f32),
        pltpu.VMEM((_CH,), i32), pltpu.VMEM((_CH,), i32),
        pltpu.VMEM((_CH,), i32), pltpu.VMEM((_CH,), i32),
        pltpu.VMEM((64,), i32),
        pltpu.VMEM((128,), f32),
        pltpu.VMEM((128,), f32),
        pltpu.VMEM((_CH,), f32), pltpu.VMEM((_CH,), f32),
        pltpu.VMEM((_CH,), f32), pltpu.VMEM((_CH,), f32),
        pltpu.VMEM((_CH,), i32), pltpu.VMEM((_CH,), i32),
        pltpu.VMEM((_CH,), f32), pltpu.VMEM((_CH,), f32),
        pltpu.VMEM((_CH,), f32), pltpu.VMEM((_CH,), f32),
        pltpu.SemaphoreType.DMA, pltpu.SemaphoreType.DMA,
        pltpu.SemaphoreType.DMA, pltpu.SemaphoreType.DMA,
    ]
    fn = pl.kernel(_sc_body, mesh=mesh, out_type=out_type, scratch_types=scratch,
                   compiler_params=pltpu.CompilerParams(needs_layout_passes=False))
    return fn(u, v, batch, time_step, la_pad, l1_pad)


def _tc_body(vi_ref, uv_ref, ut_ref, c1_ref, lai_ref, l1i_ref,
             vp_ref, lnvt_ref, lv0_ref):
    BI = vi_ref.shape[0]
    vi = vi_ref[...]
    u_v = uv_ref[...]
    u_top = ut_ref[...]
    c1 = c1_ref[...]
    la = lai_ref[...]
    l1ma = l1i_ref[...]

    z = u_v[0:1, 0:1] * 0.0
    neg30 = jnp.log(z + 1e-30)
    pos = jnp.log(z + 1.0)

    b = l1ma - _LOG_K
    a_hit = pos + la
    m_h = jnp.maximum(a_hit, b)
    q_hit = m_h + jnp.log(jnp.exp(a_hit - m_h) + jnp.exp(b - m_h))
    a_miss = neg30 + la
    m_m = jnp.maximum(a_miss, b)
    q_miss = m_m + jnp.log(jnp.exp(a_miss - m_m) + jnp.exp(b - m_m))

    g_v = -jnp.log(-jnp.log(u_v + 1e-30) + 1e-30)
    g_1 = -jnp.log(-jnp.log(u_top + 1e-30) + 1e-30)
    s_v = g_v + q_hit
    s_1 = g_1 + q_miss

    widx = jnp.where(s_v > s_1, vi,
                     jnp.where(s_1 > s_v, c1, jnp.minimum(vi, c1)))

    iota_c = lax.broadcasted_iota(jnp.int32, (BI, 128, 64), 2)
    eq_w = iota_c == widx[:, :, None]
    oh_v = iota_c == vi[:, :, None]
    pos3 = pos[:, :, None]
    neg3 = neg30[:, :, None]
    vp_ref[...] = eq_w.astype(jnp.float32)
    lnvt_ref[...] = jnp.where(eq_w, pos3, neg3)
    lv0_ref[...] = jnp.where(oh_v, pos3, neg3)


def kernel(v, time_step, batch, u, log_alphas_bar, log_1_min_alphas_bar):
    N, C = u.shape
    T = log_alphas_bar.shape[0]
    la_pad = jnp.zeros((128,), jnp.float32).at[:T].set(log_alphas_bar)
    l1_pad = jnp.zeros((128,), jnp.float32).at[:T].set(log_1_min_alphas_bar)

    u_v, u_top, c1, la_i, l1_i = _sc_stage(u, v, batch, time_step,
                                           la_pad, l1_pad)

    BI = 32
    NR = N // 128
    G = NR // BI
    col = lambda x: x.reshape(NR, 128)
    col_spec = pl.BlockSpec((BI, 128), lambda i: (i, 0))
    out_spec = pl.BlockSpec((BI, 128, C), lambda i: (i, 0, 0))
    out_sds = jax.ShapeDtypeStruct((NR, 128, C), jnp.float32)

    vp, lnvt, lv0 = pl.pallas_call(
        _tc_body,
        grid=(G,),
        in_specs=[col_spec] * 6,
        out_specs=[out_spec, out_spec, out_spec],
        out_shape=[out_sds, out_sds, out_sds],
    )(col(v), col(u_v), col(u_top), col(c1), col(la_i), col(l1_i))
    return (vp.reshape(N, C), lnvt.reshape(N, C), lv0.reshape(N, C))
